# baseline (speedup 1.0000x reference)
import functools

import jax
import jax.numpy as jnp
from jax import lax
from jax.experimental import pallas as pl
from jax.experimental.pallas import tpu as pltpu
from jax.experimental.pallas import tpu_sc as plsc

_N = 10000
_E = 320000
_D = 128
_NC = 2
_NS = 16
_NW = _NC * _NS
_C = 80
_K = 128
_G = 8
_EW = _K * _C
_EPAD = _NW * _EW
_NPAD = 10112
_RPT = _NPAD // _NS

_MESH = dict(core_axis_name="c", subcore_axis_name="s")
_MBLK = 2000


def _sc_degree(idx_hbm_arr):
    @functools.partial(
        pl.kernel,
        out_type=jax.ShapeDtypeStruct((_NC, _NPAD, 16), jnp.float32),
        mesh=plsc.VectorSubcoreMesh(**_MESH),
        scratch_types=[
            pltpu.VMEM_SHARED((_NPAD, 16), jnp.float32),
            pltpu.VMEM((2, _G, _C), jnp.int32),
            pltpu.VMEM((_C, 16), jnp.float32),
            pltpu.VMEM((128, 16), jnp.float32),
            [pltpu.SemaphoreType.DMA for _ in range(2)],
        ],
    )
    def run(idx_hbm, out_hbm, deg_sh, idx8, ones_v, zeros_v, sem):
        cid = lax.axis_index("c")
        sid = lax.axis_index("s")
        wid = cid * _NS + sid

        @pl.loop(0, _C)
        def _(i):
            ones_v[i, pl.ds(0, 16)] = jnp.ones((16,), jnp.float32)

        @pl.loop(0, 128)
        def _(i):
            zeros_v[i, pl.ds(0, 16)] = jnp.zeros((16,), jnp.float32)

        zb = sid * _RPT
        for zo in range(0, 512, 128):
            pltpu.sync_copy(zeros_v, deg_sh.at[pl.ds(zb + zo, 128)])
        pltpu.sync_copy(zeros_v.at[pl.ds(0, 120)],
                        deg_sh.at[pl.ds(zb + 512, 120)])
        plsc.subcore_barrier()

        @pl.loop(0, _K // _G)
        def _(j):
            pltpu.sync_copy(idx_hbm.at[wid, j], idx8)
            for b in range(0, _G, 2):
                s0 = pltpu.async_copy(ones_v, deg_sh.at[idx8.at[1, b]],
                                      sem[0])
                s1 = pltpu.async_copy(ones_v, deg_sh.at[idx8.at[1, b + 1]],
                                      sem[1])
                s0.wait()
                s1.wait()

        plsc.subcore_barrier()
        pltpu.sync_copy(deg_sh.at[pl.ds(zb, _RPT)],
                        out_hbm.at[cid, pl.ds(zb, _RPT)])

    return run(idx_hbm_arr)


def _sc_aggregate(hp, idx_hbm_arr):
    @functools.partial(
        pl.kernel,
        out_type=jax.ShapeDtypeStruct((_NC, _NPAD, _D), jnp.float32),
        mesh=plsc.VectorSubcoreMesh(**_MESH),
        scratch_types=[
            pltpu.VMEM_SHARED((_NPAD, _D), jnp.float32),
            pltpu.VMEM((2, _G, _C), jnp.int32),
            [pltpu.VMEM((_C, _D), jnp.float32) for _ in range(2)],
            [pltpu.SemaphoreType.DMA for _ in range(2)],
            [pltpu.SemaphoreType.DMA for _ in range(2)],
        ],
    )
    def run(hp_hbm, idx_hbm, out_hbm, acc_sh, idx8, rows, gsem, ssem):
        cid = lax.axis_index("c")
        sid = lax.axis_index("s")
        wid = cid * _NS + sid

        @pl.loop(0, _C)
        def _(i):
            @pl.loop(0, _D, step=16)
            def _(j):
                rows[0][i, pl.ds(j, 16)] = jnp.zeros((16,), jnp.float32)

        zb = sid * _RPT
        for zo in range(0, 560, _C):
            pltpu.sync_copy(rows[0], acc_sh.at[pl.ds(zb + zo, _C)])
        pltpu.sync_copy(rows[0].at[pl.ds(0, 72)],
                        acc_sh.at[pl.ds(zb + 560, 72)])
        plsc.subcore_barrier()

        @pl.loop(0, _K // _G)
        def _(j):
            pltpu.sync_copy(idx_hbm.at[wid, j], idx8)
            for b in range(0, _G, 2):
                g0 = pltpu.async_copy(hp_hbm.at[idx8.at[0, b]], rows[0],
                                      gsem[0])
                g1 = pltpu.async_copy(hp_hbm.at[idx8.at[0, b + 1]], rows[1],
                                      gsem[1])
                g0.wait()
                s0 = pltpu.async_copy(rows[0], acc_sh.at[idx8.at[1, b]],
                                      ssem[0])
                g1.wait()
                s1 = pltpu.async_copy(rows[1], acc_sh.at[idx8.at[1, b + 1]],
                                      ssem[1])
                s0.wait()
                s1.wait()

        plsc.subcore_barrier()
        pltpu.sync_copy(acc_sh.at[pl.ds(zb, _RPT)],
                        out_hbm.at[cid, pl.ds(zb, _RPT)])

    return run(hp, idx_hbm_arr)


def _dinv_from(deg_ref):
    d = deg_ref[...]
    return lax.rsqrt(d[0, :, 0] + d[1, :, 0] + 1.0)


def _tc1_body(deg_ref, x_ref, w_ref, out_ref):
    dinv = _dinv_from(deg_ref)
    h = jnp.dot(x_ref[...], w_ref[...], preferred_element_type=jnp.float32)
    out_ref[...] = h * dinv[:, None]


def _tc2_body(deg_ref, p_ref, hp_ref, b_ref, w_ref, out_ref):
    dinv = _dinv_from(deg_ref)
    p = p_ref[...]
    s = p[0] + p[1] + hp_ref[...]
    t = jnp.maximum(s * dinv[:, None] + b_ref[...], 0.0)
    h = jnp.dot(t, w_ref[...], preferred_element_type=jnp.float32)
    out_ref[...] = h * dinv[:, None]


def _tc3_body(deg_ref, p_ref, hp_ref, b_ref, w_ref, bfc_ref, out_ref):
    dinv = _dinv_from(deg_ref)
    p = p_ref[...]
    s = p[0] + p[1] + hp_ref[...]
    t = jnp.maximum(s * dinv[:, None] + b_ref[...], 0.0)
    logits = jnp.dot(t, w_ref[...], preferred_element_type=jnp.float32) + bfc_ref[...]
    m = jnp.max(logits, axis=1, keepdims=True)
    lse = jnp.log(jnp.sum(jnp.exp(logits - m), axis=1, keepdims=True)) + m
    out_ref[...] = logits - lse


_DEG_SPEC = pl.BlockSpec((_NC, _MBLK, 16), lambda i: (0, i, 0))
_ROW_SPEC = pl.BlockSpec((_MBLK, _D), lambda i: (i, 0))
_P_SPEC = pl.BlockSpec((_NC, _MBLK, _D), lambda i: (0, i, 0))
_W_SPEC = pl.BlockSpec((_D, _D), lambda i: (0, 0))
_B_SPEC = pl.BlockSpec((1, _D), lambda i: (0, 0))
_GRID = (_N // _MBLK,)
_OUT = jax.ShapeDtypeStruct((_N, _D), jnp.float32)


def _tc1(deg_p, x, w1):
    return pl.pallas_call(
        _tc1_body, grid=_GRID,
        in_specs=[_DEG_SPEC, _ROW_SPEC, _W_SPEC],
        out_specs=_ROW_SPEC, out_shape=_OUT,
    )(deg_p, x, w1)


def _tc2(deg_p, p1, hp, b, w):
    return pl.pallas_call(
        _tc2_body, grid=_GRID,
        in_specs=[_DEG_SPEC, _P_SPEC, _ROW_SPEC, _B_SPEC, _W_SPEC],
        out_specs=_ROW_SPEC, out_shape=_OUT,
    )(deg_p, p1, hp, b, w)


def _tc3(deg_p, p2, hp, b, w, bfc):
    return pl.pallas_call(
        _tc3_body, grid=_GRID,
        in_specs=[_DEG_SPEC, _P_SPEC, _ROW_SPEC, _B_SPEC, _W_SPEC, _B_SPEC],
        out_specs=_ROW_SPEC, out_shape=_OUT,
    )(deg_p, p2, hp, b, w, bfc)


def kernel(x, edge_index, W1, b1, W2, b2, Wfc, bfc):
    pad = _EPAD - _E
    src_r = jnp.concatenate(
        [edge_index[0], jnp.zeros((pad,), jnp.int32)]).reshape(
            _NW, _K // _G, _G, _C)
    dst_r = jnp.concatenate(
        [edge_index[1], jnp.full((pad,), _N, jnp.int32)]).reshape(
            _NW, _K // _G, _G, _C)
    idx_packed = jnp.stack([src_r, dst_r], axis=2)
    b1r = b1.reshape(1, _D)
    b2r = b2.reshape(1, _D)
    bfcr = bfc.reshape(1, _D)

    deg_p = _sc_degree(idx_packed)
    h1p = _tc1(deg_p, x, W1)
    p1 = _sc_aggregate(h1p, idx_packed)
    h2p = _tc2(deg_p, p1, h1p, b1r, W2)
    p2 = _sc_aggregate(h2p, idx_packed)
    return _tc3(deg_p, p2, h2p, b2r, Wfc, bfcr)


# speedup vs baseline: 1.1909x; 1.1909x over previous
import functools

import jax
import jax.numpy as jnp
from jax import lax
from jax.experimental import pallas as pl
from jax.experimental.pallas import tpu as pltpu
from jax.experimental.pallas import tpu_sc as plsc

_N = 10000
_E = 320000
_D = 128
_NC = 2
_NS = 16
_NW = _NC * _NS
_C = 128
_K = 80
_G = 8
_EW = _K * _C
_EPAD = _NW * _EW
_NPAD = 10112
_RPT = _NPAD // _NS

_MESH = dict(core_axis_name="c", subcore_axis_name="s")
_MBLK = 2000


def _sc_degree(idx_hbm_arr):
    @functools.partial(
        pl.kernel,
        out_type=jax.ShapeDtypeStruct((_NC, _NPAD, 16), jnp.float32),
        mesh=plsc.VectorSubcoreMesh(**_MESH),
        scratch_types=[
            pltpu.VMEM_SHARED((_NPAD, 16), jnp.float32),
            pltpu.VMEM((2, _G, _C), jnp.int32),
            pltpu.VMEM((_C, 16), jnp.float32),
            pltpu.VMEM((128, 16), jnp.float32),
            [pltpu.SemaphoreType.DMA for _ in range(2)],
        ],
    )
    def run(idx_hbm, out_hbm, deg_sh, idx8, ones_v, zeros_v, sem):
        cid = lax.axis_index("c")
        sid = lax.axis_index("s")
        wid = cid * _NS + sid

        @pl.loop(0, _C)
        def _(i):
            ones_v[i, pl.ds(0, 16)] = jnp.ones((16,), jnp.float32)

        @pl.loop(0, 128)
        def _(i):
            zeros_v[i, pl.ds(0, 16)] = jnp.zeros((16,), jnp.float32)

        zb = sid * _RPT
        for zo in range(0, 512, 128):
            pltpu.sync_copy(zeros_v, deg_sh.at[pl.ds(zb + zo, 128)])
        pltpu.sync_copy(zeros_v.at[pl.ds(0, 120)],
                        deg_sh.at[pl.ds(zb + 512, 120)])
        plsc.subcore_barrier()

        @pl.loop(0, _K // _G)
        def _(j):
            pltpu.sync_copy(idx_hbm.at[wid, j], idx8)
            for b in range(_G):
                pltpu.sync_copy(ones_v, deg_sh.at[idx8.at[1, b]], add=True)

        plsc.subcore_barrier()
        pltpu.sync_copy(deg_sh.at[pl.ds(zb, _RPT)],
                        out_hbm.at[cid, pl.ds(zb, _RPT)])

    return run(idx_hbm_arr)


def _sc_aggregate(hp, idx_hbm_arr):
    @functools.partial(
        pl.kernel,
        out_type=jax.ShapeDtypeStruct((_NC, _NPAD, _D), jnp.float32),
        mesh=plsc.VectorSubcoreMesh(**_MESH),
        scratch_types=[
            pltpu.VMEM_SHARED((_NPAD, _D), jnp.float32),
            pltpu.VMEM((2, _G, _C), jnp.int32),
            pltpu.VMEM((_C, _D), jnp.float32),
        ],
    )
    def run(hp_hbm, idx_hbm, out_hbm, acc_sh, idx8, rows):
        cid = lax.axis_index("c")
        sid = lax.axis_index("s")
        wid = cid * _NS + sid

        @pl.loop(0, _C)
        def _(i):
            @pl.loop(0, _D, step=16)
            def _(j):
                rows[i, pl.ds(j, 16)] = jnp.zeros((16,), jnp.float32)

        zb = sid * _RPT
        for zo in range(0, 512, _C):
            pltpu.sync_copy(rows, acc_sh.at[pl.ds(zb + zo, _C)])
        pltpu.sync_copy(rows.at[pl.ds(0, 120)],
                        acc_sh.at[pl.ds(zb + 512, 120)])
        plsc.subcore_barrier()

        @pl.loop(0, _K // _G)
        def _(j):
            pltpu.sync_copy(idx_hbm.at[wid, j], idx8)
            for b in range(_G):
                pltpu.sync_copy(hp_hbm.at[idx8.at[0, b]], rows)
                pltpu.sync_copy(rows, acc_sh.at[idx8.at[1, b]], add=True)

        plsc.subcore_barrier()
        pltpu.sync_copy(acc_sh.at[pl.ds(zb, _RPT)],
                        out_hbm.at[cid, pl.ds(zb, _RPT)])

    return run(hp, idx_hbm_arr)


def _dinv_from(deg_ref):
    d = deg_ref[...]
    return lax.rsqrt(d[0, :, 0] + d[1, :, 0] + 1.0)


def _tc1_body(deg_ref, x_ref, w_ref, out_ref):
    dinv = _dinv_from(deg_ref)
    h = jnp.dot(x_ref[...], w_ref[...], preferred_element_type=jnp.float32)
    out_ref[...] = h * dinv[:, None]


def _tc2_body(deg_ref, p_ref, hp_ref, b_ref, w_ref, out_ref):
    dinv = _dinv_from(deg_ref)
    p = p_ref[...]
    s = p[0] + p[1] + hp_ref[...]
    t = jnp.maximum(s * dinv[:, None] + b_ref[...], 0.0)
    h = jnp.dot(t, w_ref[...], preferred_element_type=jnp.float32)
    out_ref[...] = h * dinv[:, None]


def _tc3_body(deg_ref, p_ref, hp_ref, b_ref, w_ref, bfc_ref, out_ref):
    dinv = _dinv_from(deg_ref)
    p = p_ref[...]
    s = p[0] + p[1] + hp_ref[...]
    t = jnp.maximum(s * dinv[:, None] + b_ref[...], 0.0)
    logits = jnp.dot(t, w_ref[...], preferred_element_type=jnp.float32) + bfc_ref[...]
    m = jnp.max(logits, axis=1, keepdims=True)
    lse = jnp.log(jnp.sum(jnp.exp(logits - m), axis=1, keepdims=True)) + m
    out_ref[...] = logits - lse


_DEG_SPEC = pl.BlockSpec((_NC, _MBLK, 16), lambda i: (0, i, 0))
_ROW_SPEC = pl.BlockSpec((_MBLK, _D), lambda i: (i, 0))
_P_SPEC = pl.BlockSpec((_NC, _MBLK, _D), lambda i: (0, i, 0))
_W_SPEC = pl.BlockSpec((_D, _D), lambda i: (0, 0))
_B_SPEC = pl.BlockSpec((1, _D), lambda i: (0, 0))
_GRID = (_N // _MBLK,)
_OUT = jax.ShapeDtypeStruct((_N, _D), jnp.float32)


def _tc1(deg_p, x, w1):
    return pl.pallas_call(
        _tc1_body, grid=_GRID,
        in_specs=[_DEG_SPEC, _ROW_SPEC, _W_SPEC],
        out_specs=_ROW_SPEC, out_shape=_OUT,
    )(deg_p, x, w1)


def _tc2(deg_p, p1, hp, b, w):
    return pl.pallas_call(
        _tc2_body, grid=_GRID,
        in_specs=[_DEG_SPEC, _P_SPEC, _ROW_SPEC, _B_SPEC, _W_SPEC],
        out_specs=_ROW_SPEC, out_shape=_OUT,
    )(deg_p, p1, hp, b, w)


def _tc3(deg_p, p2, hp, b, w, bfc):
    return pl.pallas_call(
        _tc3_body, grid=_GRID,
        in_specs=[_DEG_SPEC, _P_SPEC, _ROW_SPEC, _B_SPEC, _W_SPEC, _B_SPEC],
        out_specs=_ROW_SPEC, out_shape=_OUT,
    )(deg_p, p2, hp, b, w, bfc)


def kernel(x, edge_index, W1, b1, W2, b2, Wfc, bfc):
    ppw = (_EPAD - _E) // _NW
    rpw = _E // _NW
    pad_src = jnp.zeros((_NW, ppw), jnp.int32)
    pad_dst = jnp.broadcast_to(
        _N + (jnp.arange(ppw, dtype=jnp.int32) % (_NPAD - _N)), (_NW, ppw))
    src_r = jnp.concatenate(
        [edge_index[0].reshape(_NW, rpw), pad_src], axis=1).reshape(
            _NW, _K // _G, _G, _C)
    dst_r = jnp.concatenate(
        [edge_index[1].reshape(_NW, rpw), pad_dst], axis=1).reshape(
            _NW, _K // _G, _G, _C)
    idx_packed = jnp.stack([src_r, dst_r], axis=2)
    b1r = b1.reshape(1, _D)
    b2r = b2.reshape(1, _D)
    bfcr = bfc.reshape(1, _D)

    deg_p = _sc_degree(idx_packed)
    h1p = _tc1(deg_p, x, W1)
    p1 = _sc_aggregate(h1p, idx_packed)
    h2p = _tc2(deg_p, p1, h1p, b1r, W2)
    p2 = _sc_aggregate(h2p, idx_packed)
    return _tc3(deg_p, p2, h2p, b2r, Wfc, bfcr)
